# baseline (speedup 1.0000x reference)
import jax
import jax.numpy as jnp
from jax.experimental import pallas as pl

N_F = 128
N_B = 16
N_S = 64 * 64


def _stats_kernel(x_ref, sum_ref, xx_ref, cnt_ref, min_ref, max_ref):
    b = pl.program_id(0)
    x = x_ref[0]
    mask = jnp.isnan(x)
    xm = jnp.where(mask, 0.0, x)

    psum = jnp.sum(xm, axis=1)[None, :]
    pcnt = jnp.sum(jnp.where(mask, 0.0, 1.0), axis=1)[None, :]
    pmin = jnp.min(jnp.where(mask, jnp.inf, x), axis=1)[None, :]
    pmax = jnp.max(jnp.where(mask, -jnp.inf, x), axis=1)[None, :]
    pxx = jax.lax.dot_general(
        xm, xm, (((1,), (1,)), ((), ())), preferred_element_type=jnp.float32
    )

    @pl.when(b == 0)
    def _init():
        sum_ref[...] = psum
        cnt_ref[...] = pcnt
        min_ref[...] = pmin
        max_ref[...] = pmax
        xx_ref[...] = pxx

    @pl.when(b != 0)
    def _acc():
        sum_ref[...] += psum
        cnt_ref[...] += pcnt
        min_ref[...] = jnp.minimum(min_ref[...], pmin)
        max_ref[...] = jnp.maximum(max_ref[...], pmax)
        xx_ref[...] += pxx


def kernel(x):
    xr = x.reshape(N_B, N_F, N_S)
    vec = jax.ShapeDtypeStruct((1, N_F), jnp.float32)
    out = pl.pallas_call(
        _stats_kernel,
        grid=(N_B,),
        in_specs=[pl.BlockSpec((1, N_F, N_S), lambda b: (b, 0, 0))],
        out_specs=[
            pl.BlockSpec((1, N_F), lambda b: (0, 0)),
            pl.BlockSpec((N_F, N_F), lambda b: (0, 0)),
            pl.BlockSpec((1, N_F), lambda b: (0, 0)),
            pl.BlockSpec((1, N_F), lambda b: (0, 0)),
            pl.BlockSpec((1, N_F), lambda b: (0, 0)),
        ],
        out_shape=[
            vec,
            jax.ShapeDtypeStruct((N_F, N_F), jnp.float32),
            vec,
            vec,
            vec,
        ],
    )(xr)
    x_sum, xx_sum, counts, min_vals, max_vals = out
    return (
        x_sum.reshape(N_F),
        xx_sum,
        counts.reshape(N_F),
        min_vals.reshape(N_F),
        max_vals.reshape(N_F),
    )
